# baseline (speedup 1.0000x reference)
import functools

import jax
import jax.numpy as jnp
from jax.experimental import pallas as pl
from jax.experimental.pallas import tpu as pltpu

_THIRD = 1.0 / 3.0
_BB = 4


def _loss_kernel(x_ref, out_ref, acc, acc_g, acc_h, *, n_rows, n_steps):
    pi = pl.program_id(0)

    @pl.when(pi == 0)
    def _init():
        acc[0:1, :] = jnp.zeros((1, acc.shape[1]), jnp.float32)
        acc[1:2, :] = jnp.full((1, acc.shape[1]), -jnp.inf, jnp.float32)
        acc_g[...] = jnp.zeros_like(acc_g)
        acc_h[...] = jnp.zeros_like(acc_h)

    dn = (((0,), (0,)), ((), ()))
    for i in range(_BB):
        x = x_ref[i]
        a = jnp.exp(-x)
        sig = 1.0 / (1.0 + a)
        om = a * sig
        s2 = sig * sig
        a3 = s2 * sig
        o2 = om * om
        b3 = o2 * om
        a3b = a3.astype(jnp.bfloat16)
        b3b = b3.astype(jnp.bfloat16)
        acc_g[...] += jax.lax.dot_general(
            a3b, b3b, dn, preferred_element_type=jnp.float32)
        acc_h[...] += jax.lax.dot_general(
            a3b, a3b, dn, preferred_element_type=jnp.float32)
        acc[0:1, :] += jnp.sum(a3, axis=0, keepdims=True)
        acc[1:2, :] = jnp.maximum(acc[1:2, :],
                                  jnp.max(x, axis=0, keepdims=True))

    @pl.when(pi == n_steps - 1)
    def _finalize():
        nc = acc.shape[1]
        inv_n = 1.0 / n_rows
        rows = jax.lax.broadcasted_iota(jnp.int32, (nc, nc), 0)
        cols = jax.lax.broadcasted_iota(jnp.int32, (nc, nc), 1)
        q1v = jnp.sum(jnp.where(rows == cols + 10, acc_g[...], 0.0),
                      axis=0, keepdims=True)
        q2v = jnp.sum(jnp.where(rows == cols + 20, acc_g[...], 0.0),
                      axis=0, keepdims=True)
        ev = jnp.sum(jnp.where(rows + 10 == cols, acc_h[...], 0.0),
                     axis=0, keepdims=True)
        p3r = (acc[0:1, :] * inv_n) ** _THIRD
        q1r = (q1v * inv_n) ** _THIRD
        q2r = (q2v * inv_n) ** _THIRD
        er = (ev * inv_n) ** _THIRD
        msig = 1.0 / (1.0 + jnp.exp(-acc[1:2, :]))
        m10 = jnp.concatenate([msig[:, 10:], msig[:, :10]], axis=1)
        m20 = jnp.concatenate([msig[:, 20:], msig[:, :20]], axis=1)
        m = jnp.maximum(m10, m20)
        lane = jax.lax.broadcasted_iota(jnp.int32, (1, nc), 1)
        is_src = jnp.logical_or(lane < 10,
                                jnp.logical_and(lane >= 30, lane < 40))
        is_e = jnp.logical_or(jnp.logical_and(lane >= 20, lane < 30),
                              jnp.logical_and(lane >= 50, lane < 60))
        picked = (jnp.where(is_src,
                            0.1 * (1.0 - m) * p3r + 0.05 * (q1r + q2r), 0.0)
                  + jnp.where(is_e, 0.1 * er, 0.0))
        out_ref[...] = jnp.sum(picked, axis=1, keepdims=True)[0:1, 0:1]


def kernel(pred_scores, target_scores):
    del target_scores
    b, a, c = pred_scores.shape
    n_rows = b * a
    n_steps = b // _BB

    out = pl.pallas_call(
        functools.partial(_loss_kernel, n_rows=n_rows, n_steps=n_steps),
        grid=(n_steps,),
        in_specs=[pl.BlockSpec((_BB, a, c), lambda i: (i, 0, 0))],
        out_specs=pl.BlockSpec((1, 1), lambda i: (0, 0)),
        out_shape=jax.ShapeDtypeStruct((1, 1), jnp.float32),
        scratch_shapes=[
            pltpu.VMEM((8, c), jnp.float32),
            pltpu.VMEM((c, c), jnp.float32),
            pltpu.VMEM((c, c), jnp.float32),
        ],
    )(pred_scores)
    return out.reshape(())
